# Initial kernel scaffold; baseline (speedup 1.0000x reference)
import jax
import jax.numpy as jnp
from jax.experimental import pallas as pl


def kernel(nfeats, efeats, edge_index, W_nodes, b_nodes, W_edges, b_edges, W_attn):
    raise NotImplementedError("write your pallas kernel here")


# speedup vs baseline: 7.7349x; 7.7349x over previous
import dataclasses
import functools

import jax
import jax.numpy as jnp
from jax import lax
from jax.experimental import pallas as pl
from jax.experimental.pallas import tpu as pltpu
from jax.experimental.pallas import tpu_sc as plsc

NC = 2
NS = 16
LN = 16


def _sc_params():
    cp = pltpu.CompilerParams()
    if "needs_layout_passes" in pltpu.CompilerParams.__dataclass_fields__:
        cp = dataclasses.replace(cp, needs_layout_passes=False)
    return cp


def _mm_cat_kernel(x_ref, w_ref, b_ref, ps_ref, pd_ref, nt_ref):
    p = lax.dot_general(x_ref[...], w_ref[...], (((1,), (0,)), ((), ())),
                        preferred_element_type=jnp.float32,
                        precision=lax.Precision.HIGHEST)
    p = p + b_ref[...]
    ps_ref[...] = p[:, 0:512]
    pd_ref[...] = p[:, 512:1024]
    nt_ref[...] = p[:, 1024:1536]


def _mm_edge_kernel(x_ref, w_ref, b_ref, o_ref):
    p = lax.dot_general(x_ref[...], w_ref[...], (((1,), (0,)), ((), ())),
                        preferred_element_type=jnp.float32,
                        precision=lax.Precision.HIGHEST)
    o_ref[...] = p + b_ref[...]


def _lane_iota():
    return lax.broadcasted_iota(jnp.int32, (LN,), 0)


def _bcast_lane(v, lane):
    s = jnp.sum(jnp.where(_lane_iota() == lane, v, 0.0))
    return jnp.zeros((LN,), jnp.float32) + s


def _edge_sc_kernel(E, N):
    EPT = E // (NC * NS)
    CB = 40
    NCHUNK = EPT // CB
    mesh = plsc.VectorSubcoreMesh(core_axis_name="c", subcore_axis_name="s")

    @functools.partial(
        pl.kernel,
        out_type=(
            jax.ShapeDtypeStruct((E, 512), jnp.float32),
            jax.ShapeDtypeStruct((E, 16), jnp.float32),
        ),
        mesh=mesh,
        compiler_params=_sc_params(),
        scratch_types=[
            pltpu.VMEM((CB,), jnp.int32),
            pltpu.VMEM((CB,), jnp.int32),
            pltpu.VMEM((CB, 512), jnp.float32),
            pltpu.VMEM((CB, 512), jnp.float32),
            pltpu.VMEM((CB, 512), jnp.float32),
            pltpu.VMEM((CB, 16), jnp.float32),
            pltpu.VMEM((64,), jnp.float32),
            pltpu.SemaphoreType.DMA,
            pltpu.SemaphoreType.DMA,
        ],
    )
    def k(psrc, pdst, eproj, src, dst, wsum, fout, arow,
          srcb, dstb, psb, pdb, epb, arb, wsb, sem0, sem1):
        cid = lax.axis_index("c")
        sid = lax.axis_index("s")
        wid = sid * NC + cid
        base = wid * EPT

        pltpu.sync_copy(wsum, wsb)
        wv = [wsb[pl.ds(m * LN, LN)] for m in range(4)]

        @pl.loop(0, NCHUNK)
        def _(i):
            e0 = base + i * CB
            pltpu.sync_copy(src.at[pl.ds(e0, CB)], srcb)
            pltpu.sync_copy(dst.at[pl.ds(e0, CB)], dstb)
            g1 = pltpu.async_copy(psrc.at[srcb], psb, sem0)
            g2 = pltpu.async_copy(pdst.at[dstb], pdb, sem1)
            pltpu.sync_copy(eproj.at[pl.ds(e0, CB), :], epb)
            g1.wait()
            g2.wait()

            @pl.loop(0, CB)
            def _(j):
                accs = [None] * 8
                for kk in range(32):
                    sl = pl.ds(kk * LN, LN)
                    v = psb[j, sl] + pdb[j, sl] + epb[j, sl]
                    f = jnp.maximum(v, 0.01 * v)
                    epb[j, sl] = f
                    h, m = kk // 4, kk % 4
                    t = f * wv[m]
                    accs[h] = t if accs[h] is None else accs[h] + t
                lanes = _lane_iota()
                z = jnp.zeros((LN,), jnp.float32)
                for h in range(8):
                    z = jnp.where(lanes == h, jnp.sum(accs[h]), z)
                ev = jnp.exp(z)
                row = jnp.where(lanes < 8, ev,
                                jnp.where(lanes == 8, 1.0, 0.0))
                arb[j] = row

            pltpu.sync_copy(epb, fout.at[pl.ds(e0, CB), :])
            pltpu.sync_copy(arb, arow.at[pl.ds(e0, CB), :])

    return k


def _denom_sc_kernel(E, N):
    EPT = E // (NC * NS)
    CB = 40
    NCHUNK = EPT // CB
    ZR = 16
    NZC = N // ZR
    NQ = (NZC + NS - 1) // NS
    mesh = plsc.VectorSubcoreMesh(core_axis_name="c", subcore_axis_name="s")

    @functools.partial(
        pl.kernel,
        out_type=(
            jax.ShapeDtypeStruct((N, 128), jnp.float32),
            jax.ShapeDtypeStruct((N, 128), jnp.float32),
        ),
        mesh=mesh,
        compiler_params=_sc_params(),
        scratch_types=[
            pltpu.VMEM((CB,), jnp.int32),
            pltpu.VMEM((CB, 16), jnp.float32),
            pltpu.VMEM((CB, 128), jnp.float32),
            pltpu.VMEM((ZR, 128), jnp.float32),
            pltpu.VMEM_SHARED((N, 128), jnp.float32),
        ],
    )
    def k(arow, dst, den0, den1, dstb, arb, vb, zb, acc):
        cid = lax.axis_index("c")
        sid = lax.axis_index("s")
        wid = sid * NC + cid
        base = wid * EPT

        @pl.loop(0, ZR)
        def _(m):
            for kk in range(8):
                zb[m, pl.ds(kk * LN, LN)] = jnp.zeros((LN,), jnp.float32)

        @pl.loop(0, CB)
        def _(m):
            for kk in range(8):
                vb[m, pl.ds(kk * LN, LN)] = jnp.zeros((LN,), jnp.float32)

        @pl.loop(0, NQ)
        def _(q):
            cix = sid + q * NS

            @pl.when(cix < NZC)
            def _():
                pltpu.sync_copy(zb, acc.at[pl.ds(cix * ZR, ZR), :])
        plsc.subcore_barrier()

        @pl.loop(0, NCHUNK)
        def _(i):
            e0 = base + i * CB
            pltpu.sync_copy(dst.at[pl.ds(e0, CB)], dstb)
            pltpu.sync_copy(arow.at[pl.ds(e0, CB), :], arb)

            @pl.loop(0, CB)
            def _(j):
                vb[j, pl.ds(0, LN)] = arb[j]

            pltpu.sync_copy(vb, acc.at[dstb], add=True)

        plsc.subcore_barrier()

        @pl.loop(0, NQ)
        def _(q):
            cix = sid + q * NS

            @pl.when((cix < NZC) & (cid == 0))
            def _():
                sl = pl.ds(cix * ZR, ZR)
                pltpu.sync_copy(acc.at[sl, :], den0.at[sl, :])

            @pl.when((cix < NZC) & (cid == 1))
            def _():
                sl = pl.ds(cix * ZR, ZR)
                pltpu.sync_copy(acc.at[sl, :], den1.at[sl, :])

    return k


def _agg_sc_kernel(E, N):
    EPT = E // NS
    CB = 80
    NCHUNK = EPT // CB
    ZR = 16
    NZC = N // ZR
    NQ = (NZC + NS - 1) // NS
    mesh = plsc.VectorSubcoreMesh(core_axis_name="c", subcore_axis_name="s")

    @functools.partial(
        pl.kernel,
        out_type=jax.ShapeDtypeStruct((4, N, 128), jnp.float32),
        mesh=mesh,
        compiler_params=_sc_params(),
        scratch_types=[
            pltpu.VMEM((CB,), jnp.int32),
            pltpu.VMEM((CB,), jnp.int32),
            pltpu.VMEM((CB,), jnp.int32),
            pltpu.VMEM((CB, 128), jnp.float32),
            pltpu.VMEM((CB, 16), jnp.float32),
            pltpu.VMEM((ZR, 128), jnp.float32),
            pltpu.VMEM((ZR, 128), jnp.float32),
            pltpu.VMEM((ZR, 128), jnp.float32),
            pltpu.VMEM((ZR, 128), jnp.float32),
            pltpu.VMEM_SHARED((N, 128), jnp.float32),
            pltpu.SemaphoreType.DMA,
        ],
    )
    def k(ntg, arow, src, dst, den0, den1, hnew,
          srcb, dstb, idxb, rowb, arb, zb, ntb, q0, q1,
          acc, sem0):
        cid = lax.axis_index("c")
        sid = lax.axis_index("s")

        for gg in range(2):
            g = cid * 2 + gg

            @pl.loop(0, ZR)
            def _(m):
                for kk in range(8):
                    zb[m, pl.ds(kk * LN, LN)] = jnp.zeros((LN,), jnp.float32)

            @pl.loop(0, NQ)
            def _(q):
                cix = sid + q * NS

                @pl.when(cix < NZC)
                def _():
                    pltpu.sync_copy(zb, acc.at[pl.ds(cix * ZR, ZR), :])
            plsc.subcore_barrier()

            @pl.loop(0, NCHUNK)
            def _(i):
                e0 = sid * EPT + i * CB
                pltpu.sync_copy(src.at[pl.ds(e0, CB)], srcb)
                pltpu.sync_copy(dst.at[pl.ds(e0, CB)], dstb)

                @pl.loop(0, CB // LN)
                def _(m):
                    sl = pl.ds(m * LN, LN)
                    idxb[sl] = srcb[sl] + g * N

                g1 = pltpu.async_copy(ntg.at[idxb], rowb, sem0)
                pltpu.sync_copy(arow.at[pl.ds(e0, CB), :], arb)
                g1.wait()

                @pl.loop(0, CB)
                def _(j):
                    crow = arb[j]
                    c0 = _bcast_lane(crow, 2 * g)
                    c1 = _bcast_lane(crow, 2 * g + 1)
                    for kk in range(8):
                        sl = pl.ds(kk * LN, LN)
                        cv = c0 if kk < 4 else c1
                        rowb[j, sl] = rowb[j, sl] * cv

                pltpu.sync_copy(rowb, acc.at[dstb], add=True)

            plsc.subcore_barrier()

            @pl.loop(0, NQ)
            def _(q):
                cix = sid + q * NS

                @pl.when(cix < NZC)
                def _():
                    n0 = cix * ZR
                    pltpu.sync_copy(acc.at[pl.ds(n0, ZR), :], zb)
                    pltpu.sync_copy(ntg.at[pl.ds(g * N + n0, ZR), :], ntb)
                    pltpu.sync_copy(den0.at[pl.ds(n0, ZR), :], q0)
                    pltpu.sync_copy(den1.at[pl.ds(n0, ZR), :], q1)

                    @pl.loop(0, ZR)
                    def _(m):
                        dv = q0[m, pl.ds(0, LN)] + q1[m, pl.ds(0, LN)]
                        pos = _bcast_lane(dv, 8) > 0.0
                        d0 = _bcast_lane(dv, 2 * g)
                        d1 = _bcast_lane(dv, 2 * g + 1)
                        for kk in range(8):
                            sl = pl.ds(kk * LN, LN)
                            dd = d0 if kk < 4 else d1
                            zb[m, sl] = jnp.where(pos, zb[m, sl] / dd,
                                                  ntb[m, sl])

                    pltpu.sync_copy(zb, hnew.at[g, pl.ds(n0, ZR), :])
            plsc.subcore_barrier()

    return k


def kernel(nfeats, efeats, edge_index, W_nodes, b_nodes, W_edges, b_edges, W_attn):
    N, D_IN = nfeats.shape
    E = efeats.shape[0]
    H = W_attn.shape[0]
    DO = W_attn.shape[1]

    src = edge_index[0]
    dst = edge_index[1]

    w_cat = jnp.concatenate(
        [W_edges[:, :D_IN].T, W_edges[:, D_IN + 16:].T, W_nodes.T], axis=1)
    b_cat = jnp.concatenate(
        [b_edges, b_edges, b_nodes], axis=0).reshape(1, 3 * H * DO)
    w_ef = W_edges[:, D_IN:D_IN + 16].T
    b_ef = b_edges.reshape(1, H * DO)
    w_sum = jnp.sum(W_attn, axis=0)

    NB = 400
    psrc, pdst, nt = pl.pallas_call(
        _mm_cat_kernel,
        grid=(N // NB,),
        in_specs=[
            pl.BlockSpec((NB, D_IN), lambda i: (i, 0)),
            pl.BlockSpec((D_IN, 3 * H * DO), lambda i: (0, 0)),
            pl.BlockSpec((1, 3 * H * DO), lambda i: (0, 0)),
        ],
        out_specs=[
            pl.BlockSpec((NB, H * DO), lambda i: (i, 0)),
            pl.BlockSpec((NB, H * DO), lambda i: (i, 0)),
            pl.BlockSpec((NB, H * DO), lambda i: (i, 0)),
        ],
        out_shape=[
            jax.ShapeDtypeStruct((N, H * DO), jnp.float32),
            jax.ShapeDtypeStruct((N, H * DO), jnp.float32),
            jax.ShapeDtypeStruct((N, H * DO), jnp.float32),
        ],
    )(nfeats, w_cat, b_cat)

    EB = 2000
    eproj = pl.pallas_call(
        _mm_edge_kernel,
        grid=(E // EB,),
        in_specs=[
            pl.BlockSpec((EB, 16), lambda i: (i, 0)),
            pl.BlockSpec((16, H * DO), lambda i: (0, 0)),
            pl.BlockSpec((1, H * DO), lambda i: (0, 0)),
        ],
        out_specs=pl.BlockSpec((EB, H * DO), lambda i: (i, 0)),
        out_shape=jax.ShapeDtypeStruct((E, H * DO), jnp.float32),
    )(efeats, w_ef, b_ef)

    fout, arow = _edge_sc_kernel(E, N)(psrc, pdst, eproj, src, dst, w_sum)

    den0, den1 = _denom_sc_kernel(E, N)(arow, dst)

    ntg = nt.reshape(N, 4, 128).transpose(1, 0, 2).reshape(4 * N, 128)
    hnew = _agg_sc_kernel(E, N)(ntg, arow, src, dst, den0, den1)

    return (hnew.reshape(4, N, 2, DO).transpose(1, 0, 2, 3).reshape(N, H, DO),
            fout.reshape(E, H, DO))
